# baseline (speedup 1.0000x reference)
import functools

import jax
import jax.numpy as jnp
from jax import lax
from jax.experimental import pallas as pl
from jax.experimental.pallas import tpu as pltpu
from jax.experimental.pallas import tpu_sc as plsc

_VOCAB = 100000
_HIDDEN = 1024
_BATCH = 4
_SEQ = 4096

_NW = 32
_B = _BATCH * _SEQ
_BPW = _B // _NW
_K = 8
_NBUF = 8
_A = 7
_NCHUNK = _BPW // _K


@functools.partial(
    pl.kernel,
    out_type=jax.ShapeDtypeStruct((_B, _HIDDEN), jnp.float32),
    mesh=plsc.VectorSubcoreMesh(core_axis_name="c", subcore_axis_name="s"),
    scratch_types=[
        pltpu.VMEM((_BPW,), jnp.int32),
        [pltpu.VMEM((_K, _HIDDEN), jnp.float32) for _ in range(_NBUF)],
        [pltpu.SemaphoreType.DMA for _ in range(_NBUF)],
        [pltpu.SemaphoreType.DMA for _ in range(_NBUF)],
    ],
)
def _embed(table_hbm, idx_hbm, out_hbm, idx_v, bufs, gsems, ssems):
    wid = lax.axis_index("s") * 2 + lax.axis_index("c")
    base = wid * _BPW

    pltpu.sync_copy(idx_hbm.at[pl.ds(base, _BPW)], idx_v)

    def gather(i, b):
        pltpu.async_copy(
            table_hbm.at[idx_v.at[pl.ds(i * _K, _K)]], bufs[b], gsems[b]
        )

    def wait_gather(i, b):
        pltpu.make_async_copy(
            table_hbm.at[idx_v.at[pl.ds(i * _K, _K)]], bufs[b], gsems[b]
        ).wait()

    def scatter(i, b):
        pltpu.async_copy(
            bufs[b], out_hbm.at[pl.ds(base + i * _K, _K)], ssems[b]
        )

    def wait_scatter(i, b):
        pltpu.make_async_copy(
            bufs[b], out_hbm.at[pl.ds(base + i * _K, _K)], ssems[b]
        ).wait()

    for i in range(_A):
        gather(i, i)

    def step(g, carry):
        for b in range(_NBUF):
            i = g + b
            wait_gather(i, b)
            scatter(i, b)
            ia = i + _A
            ba = (b + _A) % _NBUF

            @pl.when(ia < _NCHUNK)
            def _():
                @pl.when(ia >= _NBUF)
                def _():
                    wait_scatter(ia - _NBUF, ba)

                gather(ia, ba)

        return carry

    lax.fori_loop(
        0, _NCHUNK // _NBUF, lambda t, c: step(t * _NBUF, c), 0, unroll=False
    )

    for b in range(_NBUF):
        wait_scatter(_NCHUNK - _NBUF + b, b)


def kernel(input_, weight):
    idx = input_.reshape(-1).astype(jnp.int32)
    out = _embed(weight, idx)
    return out.reshape(_BATCH, _SEQ, _HIDDEN)


# speedup vs baseline: 1.0044x; 1.0044x over previous
import functools

import jax
import jax.numpy as jnp
from jax import lax
from jax.experimental import pallas as pl
from jax.experimental.pallas import tpu as pltpu
from jax.experimental.pallas import tpu_sc as plsc

_VOCAB = 100000
_HIDDEN = 1024
_BATCH = 4
_SEQ = 4096

_NW = 32
_B = _BATCH * _SEQ
_BPW = _B // _NW
_K = 8
_NBUF = 8
_A = 6
_NCHUNK = _BPW // _K


@functools.partial(
    pl.kernel,
    out_type=jax.ShapeDtypeStruct((_B, _HIDDEN), jnp.float32),
    mesh=plsc.VectorSubcoreMesh(core_axis_name="c", subcore_axis_name="s"),
    scratch_types=[
        pltpu.VMEM((_BPW,), jnp.int32),
        [pltpu.VMEM((_K, _HIDDEN), jnp.float32) for _ in range(_NBUF)],
        [pltpu.SemaphoreType.DMA for _ in range(_NBUF)],
        [pltpu.SemaphoreType.DMA for _ in range(_NBUF)],
    ],
)
def _embed(table_hbm, idx_hbm, out_hbm, idx_v, bufs, gsems, ssems):
    wid = lax.axis_index("s") * 2 + lax.axis_index("c")
    base = wid * _BPW

    pltpu.sync_copy(idx_hbm.at[pl.ds(base, _BPW)], idx_v)

    def gather(i, b):
        pltpu.async_copy(
            table_hbm.at[idx_v.at[pl.ds(i * _K, _K)]], bufs[b], gsems[b]
        )

    def wait_gather(i, b):
        pltpu.make_async_copy(
            table_hbm.at[idx_v.at[pl.ds(i * _K, _K)]], bufs[b], gsems[b]
        ).wait()

    def scatter(i, b):
        pltpu.async_copy(
            bufs[b], out_hbm.at[pl.ds(base + i * _K, _K)], ssems[b]
        )

    def wait_scatter(i, b):
        pltpu.make_async_copy(
            bufs[b], out_hbm.at[pl.ds(base + i * _K, _K)], ssems[b]
        ).wait()

    for i in range(_A):
        gather(i, i)

    def step(g, carry):
        for b in range(_NBUF):
            i = g + b
            wait_gather(i, b)
            scatter(i, b)
            ia = i + _A
            ba = (b + _A) % _NBUF

            @pl.when(ia < _NCHUNK)
            def _():
                @pl.when(ia >= _NBUF)
                def _():
                    wait_scatter(ia - _NBUF, ba)

                gather(ia, ba)

        return carry

    lax.fori_loop(
        0, _NCHUNK // _NBUF, lambda t, c: step(t * _NBUF, c), 0, unroll=False
    )

    for b in range(_NBUF):
        wait_scatter(_NCHUNK - _NBUF + b, b)


def kernel(input_, weight):
    idx = input_.reshape(-1).astype(jnp.int32)
    out = _embed(weight, idx)
    return out.reshape(_BATCH, _SEQ, _HIDDEN)
